# Initial kernel scaffold; baseline (speedup 1.0000x reference)
import jax
import jax.numpy as jnp
from jax.experimental import pallas as pl


def kernel(prefix, embedding_table):
    raise NotImplementedError("write your pallas kernel here")


# speedup vs baseline: 1.4488x; 1.4488x over previous
import jax
import jax.numpy as jnp
from jax.experimental import pallas as pl
from jax.experimental.pallas import tpu as pltpu

_B = 1024
_P = 20
_V = 20
_D = 18432
_BLK = 256


def _body(idx_ref, table_ref, out_ref):
    idx = idx_ref[0, 0]
    iota = jax.lax.broadcasted_iota(jnp.int32, (_BLK, _V), 1)
    onehot = (idx[:, None] == iota).astype(jnp.float32)
    out_ref[...] = jnp.dot(onehot, table_ref[...],
                           preferred_element_type=jnp.float32)


def kernel(prefix, embedding_table):
    n = _B * _P
    nblk = n // _BLK
    idx3 = prefix.reshape(nblk, 1, _BLK)
    out = pl.pallas_call(
        _body,
        grid=(nblk,),
        in_specs=[
            pl.BlockSpec((1, 1, _BLK), lambda i: (i, 0, 0)),
            pl.BlockSpec((_V, _D), lambda i: (0, 0)),
        ],
        out_specs=pl.BlockSpec((_BLK, _D), lambda i: (i, 0)),
        out_shape=jax.ShapeDtypeStruct((n, _D), jnp.float32),
    )(idx3, embedding_table)
    return out.reshape(_B, _P, _D)
